# baseline (speedup 1.0000x reference)
import jax
import jax.numpy as jnp
from jax.experimental import pallas as pl

B = 8
P = 19248
C = 81
M = 32
BPA = 6416
NBLKA = P // BPA
MROWS = B * P * M // 128
MBS = 12832
NBLKB = MROWS // MBS
LROWS = B * P * 4 // 128
NEG_POS_RATIO = 3
BBOX_W = 1.0
MASK_W = 0.2 / 32.0


def _class_body(conf_ref, lab_ref, rank_ref, ce_ref):
    conf = conf_ref[0]
    labels = lab_ref[0]
    amax = jnp.max(conf, axis=-1, keepdims=True)
    ex = jnp.exp(conf - amax)
    lse = jnp.log(jnp.sum(ex, axis=-1, keepdims=True)) + amax
    iota = jax.lax.broadcasted_iota(jnp.int32, (BPA, C), 1)
    xl = jnp.sum(jnp.where(iota == labels, conf, 0.0), axis=-1, keepdims=True)
    rank_ref[0] = lse - conf[:, 0:1]
    ce_ref[0] = lse - xl


def _elem_body(mask_ref, maskt_ref, posm_ref, loc_ref, loct_ref, posl_ref,
               st_ref):
    j = pl.program_id(0)
    p = jnp.clip(mask_ref[...], 1e-7, 1.0 - 1e-7)
    mt = maskt_ref[...]
    a = jnp.log(p)
    bb = jnp.log1p(-p)
    bce = mt * (bb - a) - bb
    l_mask = jnp.sum(bce * posm_ref[...].astype(jnp.float32))
    ones = jnp.ones((1, 128), jnp.float32)
    st_ref[0, 0:1, :] = l_mask * ones

    @pl.when(j == 0)
    def _():
        d = jnp.abs(loc_ref[...] - loct_ref[...])
        sl1 = jnp.where(d < 1.0, 0.5 * d * d, d - 0.5)
        l_loc = jnp.sum(sl1 * posl_ref[...].astype(jnp.float32))
        st_ref[0, 1:2, :] = l_loc * ones

    @pl.when(j != 0)
    def _():
        st_ref[0, 1:2, :] = jnp.zeros((1, 128), jnp.float32)


def _select_body(rank_ref, ce_ref, conft_ref, st_ref, out_ref):
    labels = conft_ref[...]
    pos = labels > 0
    inv = labels < 0
    posf = jnp.where(pos, 1.0, 0.0)

    ce = ce_ref[...]
    s = jnp.where(pos | inv, 0.0, rank_ref[...])
    cenv = jnp.where(pos | inv, 0.0, ce)

    np_rows = jnp.sum(posf, axis=1, keepdims=True)
    np_total = jnp.sum(np_rows)
    ce_pos_tot = jnp.sum(ce * posf)
    l_mask_tot = jnp.sum(st_ref[:, 0, 0:1])
    l_loc_tot = jnp.sum(st_ref[:, 1, 0:1])

    k = jnp.minimum(NEG_POS_RATIO * np_rows.astype(jnp.int32), P - 1)
    bits = jax.lax.bitcast_convert_type(s, jnp.int32)

    def t_step(i, pref):
        cand = pref | (jnp.int32(1) << (30 - i))
        cnt = jnp.sum((bits >= cand).astype(jnp.int32), axis=1, keepdims=True)
        return jnp.where(cnt >= k, cand, pref)

    t = jax.lax.fori_loop(0, 31, t_step, jnp.zeros((B, 1), jnp.int32))

    cgt = jnp.sum((bits > t).astype(jnp.int32), axis=1, keepdims=True)
    rem = k - cgt
    tie = bits == t
    idx = jax.lax.broadcasted_iota(jnp.int32, (B, P), 1)

    def j_step(i, acc):
        cand = acc | (jnp.int32(1) << (14 - i))
        cnt = jnp.sum((tie & (idx < cand)).astype(jnp.int32),
                      axis=1, keepdims=True)
        return jnp.where(cnt <= rem, cand, acc)

    j_lim = jax.lax.fori_loop(0, 15, j_step, jnp.zeros((B, 1), jnp.int32))

    sel = (bits > t) | (tie & (idx < j_lim))
    neg_sum = jnp.sum(jnp.where(sel, cenv, 0.0))

    n = jnp.maximum(np_total, 1.0)
    loss_l = l_loc_tot * BBOX_W / n
    loss_c = (ce_pos_tot + neg_sum) / n
    loss_m = l_mask_tot * MASK_W / n
    ones = jnp.ones((1, 128), jnp.float32)
    out_ref[...] = jnp.concatenate(
        [loss_l * ones, loss_c * ones, loss_m * ones,
         jnp.zeros((5, 128), jnp.float32)], axis=0)


def _run(loc_data, conf_data, mask_data, loc_t, conf_t, masks_t,
         interpret=False):
    conf_t3 = conf_t.reshape(B, P, 1)
    rank, ce = pl.pallas_call(
        _class_body,
        grid=(B, NBLKA),
        in_specs=[
            pl.BlockSpec((1, BPA, C), lambda b, j: (b, j, 0)),
            pl.BlockSpec((1, BPA, 1), lambda b, j: (b, j, 0)),
        ],
        out_specs=[
            pl.BlockSpec((1, BPA, 1), lambda b, j: (b, j, 0)),
            pl.BlockSpec((1, BPA, 1), lambda b, j: (b, j, 0)),
        ],
        out_shape=[jax.ShapeDtypeStruct((B, P, 1), jnp.float32)] * 2,
        interpret=interpret,
    )(conf_data, conf_t3)

    pos8 = (conf_t > 0).astype(jnp.uint8)[..., None]
    posm = jnp.broadcast_to(pos8, (B, P, M)).reshape(MROWS, 128)
    posl = jnp.broadcast_to(pos8, (B, P, 4)).reshape(LROWS, 128)
    full = lambda rows: pl.BlockSpec((rows, 128), lambda j: (0, 0))
    stb = pl.pallas_call(
        _elem_body,
        grid=(NBLKB,),
        in_specs=[
            pl.BlockSpec((MBS, 128), lambda j: (j, 0)),
            pl.BlockSpec((MBS, 128), lambda j: (j, 0)),
            pl.BlockSpec((MBS, 128), lambda j: (j, 0)),
            full(LROWS), full(LROWS), full(LROWS),
        ],
        out_specs=[pl.BlockSpec((1, 2, 128), lambda j: (j, 0, 0))],
        out_shape=[jax.ShapeDtypeStruct((NBLKB, 2, 128), jnp.float32)],
        interpret=interpret,
    )(mask_data.reshape(MROWS, 128), masks_t.reshape(MROWS, 128), posm,
      loc_data.reshape(LROWS, 128), loc_t.reshape(LROWS, 128), posl)[0]

    out = pl.pallas_call(
        _select_body,
        out_shape=jax.ShapeDtypeStruct((8, 128), jnp.float32),
        interpret=interpret,
    )(rank.reshape(B, P), ce.reshape(B, P), conf_t, stb)
    return (out[0, 0], out[1, 0], out[2, 0])


def kernel(loc_data, conf_data, mask_data, loc_t, conf_t, masks_t):
    return _run(loc_data, conf_data, mask_data, loc_t, conf_t, masks_t)


# speedup vs baseline: 1.0389x; 1.0389x over previous
import jax
import jax.numpy as jnp
from jax.experimental import pallas as pl

B = 8
P = 19248
C = 81
M = 32
BPA = 6416
NBLKA = P // BPA
MROWS = B * P * M // 128
MBS = 3208
NBLKB = MROWS // MBS
LROWS = B * P * 4 // 128
NEG_POS_RATIO = 3
BBOX_W = 1.0
MASK_W = 0.2 / 32.0


def _class_body(conf_ref, lab_ref, s_ref, st_ref):
    conf = conf_ref[0]
    labels = lab_ref[0]
    amax = jnp.max(conf, axis=-1, keepdims=True)
    ex = jnp.exp(conf - amax)
    lse = jnp.log(jnp.sum(ex, axis=-1, keepdims=True)) + amax
    pos = labels > 0
    skip = pos | (labels < 0)
    s_ref[0] = jnp.where(skip, 0.0, lse - conf[:, 0:1])
    iota = jax.lax.broadcasted_iota(jnp.int32, (BPA, C), 1)
    xl = jnp.sum(jnp.where(iota == labels, conf, 0.0), axis=-1, keepdims=True)
    npos = jnp.sum(jnp.where(pos, 1.0, 0.0))
    cepos = jnp.sum(jnp.where(pos, lse - xl, 0.0))
    ones = jnp.ones((1, 128), jnp.float32)
    st_ref[0, 0] = jnp.concatenate([npos * ones, cepos * ones], axis=0)


def _elem_body(mask_ref, maskt_ref, posm_ref, loc_ref, loct_ref, posl_ref,
               st_ref):
    j = pl.program_id(0)
    p = jnp.clip(mask_ref[...], 1e-7, 1.0 - 1e-7)
    mt = maskt_ref[...]
    a = jnp.log(p)
    bb = jnp.log1p(-p)
    bce = mt * (bb - a) - bb
    l_mask = jnp.sum(bce * posm_ref[...])
    ones = jnp.ones((1, 128), jnp.float32)
    st_ref[0, 0:1, :] = l_mask * ones

    @pl.when(j == 0)
    def _():
        d = jnp.abs(loc_ref[...] - loct_ref[...])
        sl1 = jnp.where(d < 1.0, 0.5 * d * d, d - 0.5)
        l_loc = jnp.sum(sl1 * posl_ref[...])
        st_ref[0, 1:2, :] = l_loc * ones

    @pl.when(j != 0)
    def _():
        st_ref[0, 1:2, :] = jnp.zeros((1, 128), jnp.float32)


def _select_body(s_ref, sta_ref, stb_ref, out_ref):
    s = s_ref[...]
    np_rows = jnp.sum(sta_ref[:, :, 0, 0:1], axis=1)
    np_total = jnp.sum(np_rows)
    ce_pos_tot = jnp.sum(sta_ref[:, :, 1, 0:1])
    l_mask_tot = jnp.sum(stb_ref[:, 0, 0:1])
    l_loc_tot = jnp.sum(stb_ref[:, 1, 0:1])

    k = jnp.minimum(NEG_POS_RATIO * np_rows.astype(jnp.int32), P - 1)
    bits = jax.lax.bitcast_convert_type(s, jnp.int32)

    def t_step(i, pref):
        cand = pref | (jnp.int32(1) << (30 - i))
        cnt = jnp.sum((bits >= cand).astype(jnp.int32), axis=1, keepdims=True)
        return jnp.where(cnt >= k, cand, pref)

    t = jax.lax.fori_loop(0, 31, t_step, jnp.zeros((B, 1), jnp.int32))

    cgt = jnp.sum((bits > t).astype(jnp.int32), axis=1, keepdims=True)
    rem = k - cgt
    tie = bits == t
    idx = jax.lax.broadcasted_iota(jnp.int32, (B, P), 1)

    def j_step(i, acc):
        cand = acc | (jnp.int32(1) << (14 - i))
        cnt = jnp.sum((tie & (idx < cand)).astype(jnp.int32),
                      axis=1, keepdims=True)
        return jnp.where(cnt <= rem, cand, acc)

    j_lim = jax.lax.fori_loop(0, 15, j_step, jnp.zeros((B, 1), jnp.int32))

    sel = (bits > t) | (tie & (idx < j_lim))
    neg_sum = jnp.sum(jnp.where(sel, s, 0.0))

    n = jnp.maximum(np_total, 1.0)
    loss_l = l_loc_tot * BBOX_W / n
    loss_c = (ce_pos_tot + neg_sum) / n
    loss_m = l_mask_tot * MASK_W / n
    ones = jnp.ones((1, 128), jnp.float32)
    out_ref[...] = jnp.concatenate(
        [loss_l * ones, loss_c * ones, loss_m * ones,
         jnp.zeros((5, 128), jnp.float32)], axis=0)


def _run(loc_data, conf_data, mask_data, loc_t, conf_t, masks_t,
         interpret=False):
    conf_t3 = conf_t.reshape(B, P, 1)
    s, sta = pl.pallas_call(
        _class_body,
        grid=(B, NBLKA),
        in_specs=[
            pl.BlockSpec((1, BPA, C), lambda b, j: (b, j, 0)),
            pl.BlockSpec((1, BPA, 1), lambda b, j: (b, j, 0)),
        ],
        out_specs=[
            pl.BlockSpec((1, BPA, 1), lambda b, j: (b, j, 0)),
            pl.BlockSpec((1, 1, 2, 128), lambda b, j: (b, j, 0, 0)),
        ],
        out_shape=[
            jax.ShapeDtypeStruct((B, P, 1), jnp.float32),
            jax.ShapeDtypeStruct((B, NBLKA, 2, 128), jnp.float32),
        ],
        interpret=interpret,
    )(conf_data, conf_t3)

    posf = (conf_t > 0).astype(jnp.float32)[..., None]
    posm = jnp.broadcast_to(posf, (B, P, M)).reshape(MROWS, 128)
    posl = jnp.broadcast_to(posf, (B, P, 4)).reshape(LROWS, 128)
    full = lambda: pl.BlockSpec((LROWS, 128), lambda j: (0, 0))
    stb = pl.pallas_call(
        _elem_body,
        grid=(NBLKB,),
        in_specs=[
            pl.BlockSpec((MBS, 128), lambda j: (j, 0)),
            pl.BlockSpec((MBS, 128), lambda j: (j, 0)),
            pl.BlockSpec((MBS, 128), lambda j: (j, 0)),
            full(), full(), full(),
        ],
        out_specs=[pl.BlockSpec((1, 2, 128), lambda j: (j, 0, 0))],
        out_shape=[jax.ShapeDtypeStruct((NBLKB, 2, 128), jnp.float32)],
        interpret=interpret,
    )(mask_data.reshape(MROWS, 128), masks_t.reshape(MROWS, 128), posm,
      loc_data.reshape(LROWS, 128), loc_t.reshape(LROWS, 128), posl)[0]

    out = pl.pallas_call(
        _select_body,
        out_shape=jax.ShapeDtypeStruct((8, 128), jnp.float32),
        interpret=interpret,
    )(s.reshape(B, P), sta, stb)
    return (out[0, 0], out[1, 0], out[2, 0])


def kernel(loc_data, conf_data, mask_data, loc_t, conf_t, masks_t):
    return _run(loc_data, conf_data, mask_data, loc_t, conf_t, masks_t)


# speedup vs baseline: 1.5950x; 1.5352x over previous
import jax
import jax.numpy as jnp
from jax.experimental import pallas as pl

B = 8
P = 19248
C = 81
M = 32
BP = 3208
NBLK = P // BP
NEG_POS_RATIO = 3
BBOX_W = 1.0
MASK_W = 0.2 / 32.0


def _pass1_body(loc_ref, loct_ref, conf_ref, conft_ref, mask_ref, maskt_ref,
                s_ref, st_ref):
    conf = conf_ref[0]
    labels = conft_ref[0]
    amax = jnp.max(conf, axis=-1, keepdims=True)
    ex = jnp.exp(conf - amax)
    lse = jnp.log(jnp.sum(ex, axis=-1, keepdims=True)) + amax
    pos = labels > 0
    skip = pos | (labels < 0)
    s_ref[0] = jnp.where(skip, 0.0, lse - conf[:, 0:1])
    iota = jax.lax.broadcasted_iota(jnp.int32, (BP, C), 1)
    xl = jnp.sum(jnp.where(iota == labels, conf, 0.0), axis=-1, keepdims=True)
    posf = jnp.where(pos, 1.0, 0.0)
    npos = jnp.sum(posf)
    cepos = jnp.sum(jnp.where(pos, lse - xl, 0.0))

    d = jnp.abs(loc_ref[0] - loct_ref[0])
    sl1 = jnp.where(d < 1.0, 0.5 * d * d, d - 0.5)
    l_loc = jnp.sum(jnp.sum(sl1, axis=-1, keepdims=True) * posf)

    p = jnp.clip(mask_ref[0], 1e-7, 1.0 - 1e-7)
    mt = maskt_ref[0]
    a = jnp.log(p)
    bb = jnp.log1p(-p)
    bce = mt * (bb - a) - bb
    l_mask = jnp.sum(jnp.sum(bce, axis=-1, keepdims=True) * posf)

    ones = jnp.ones((1, 128), jnp.float32)
    st_ref[0, 0] = jnp.concatenate(
        [npos * ones, cepos * ones, l_loc * ones, l_mask * ones], axis=0)


def _select_body(s_ref, st_ref, out_ref):
    s = s_ref[...]
    np_rows = jnp.sum(st_ref[:, :, 0, 0:1], axis=1)
    np_total = jnp.sum(np_rows)
    ce_pos_tot = jnp.sum(st_ref[:, :, 1, 0:1])
    l_loc_tot = jnp.sum(st_ref[:, :, 2, 0:1])
    l_mask_tot = jnp.sum(st_ref[:, :, 3, 0:1])

    k = jnp.minimum(NEG_POS_RATIO * np_rows.astype(jnp.int32), P - 1)
    bits = jax.lax.bitcast_convert_type(s, jnp.int32)

    def t_step(i, pref):
        cand = pref | (jnp.int32(1) << (30 - i))
        cnt = jnp.sum((bits >= cand).astype(jnp.int32), axis=1, keepdims=True)
        return jnp.where(cnt >= k, cand, pref)

    t = jax.lax.fori_loop(0, 31, t_step, jnp.zeros((B, 1), jnp.int32))

    cgt = jnp.sum((bits > t).astype(jnp.int32), axis=1, keepdims=True)
    rem = k - cgt
    tie = bits == t
    idx = jax.lax.broadcasted_iota(jnp.int32, (B, P), 1)

    def j_step(i, acc):
        cand = acc | (jnp.int32(1) << (14 - i))
        cnt = jnp.sum((tie & (idx < cand)).astype(jnp.int32),
                      axis=1, keepdims=True)
        return jnp.where(cnt <= rem, cand, acc)

    j_lim = jax.lax.fori_loop(0, 15, j_step, jnp.zeros((B, 1), jnp.int32))

    sel = (bits > t) | (tie & (idx < j_lim))
    neg_sum = jnp.sum(jnp.where(sel, s, 0.0))

    n = jnp.maximum(np_total, 1.0)
    loss_l = l_loc_tot * BBOX_W / n
    loss_c = (ce_pos_tot + neg_sum) / n
    loss_m = l_mask_tot * MASK_W / n
    ones = jnp.ones((1, 128), jnp.float32)
    out_ref[...] = jnp.concatenate(
        [loss_l * ones, loss_c * ones, loss_m * ones,
         jnp.zeros((5, 128), jnp.float32)], axis=0)


def _run(loc_data, conf_data, mask_data, loc_t, conf_t, masks_t,
         interpret=False):
    conf_t3 = conf_t.reshape(B, P, 1)
    s, sta = pl.pallas_call(
        _pass1_body,
        grid=(B, NBLK),
        in_specs=[
            pl.BlockSpec((1, BP, 4), lambda b, j: (b, j, 0)),
            pl.BlockSpec((1, BP, 4), lambda b, j: (b, j, 0)),
            pl.BlockSpec((1, BP, C), lambda b, j: (b, j, 0)),
            pl.BlockSpec((1, BP, 1), lambda b, j: (b, j, 0)),
            pl.BlockSpec((1, BP, M), lambda b, j: (b, j, 0)),
            pl.BlockSpec((1, BP, M), lambda b, j: (b, j, 0)),
        ],
        out_specs=[
            pl.BlockSpec((1, BP, 1), lambda b, j: (b, j, 0)),
            pl.BlockSpec((1, 1, 4, 128), lambda b, j: (b, j, 0, 0)),
        ],
        out_shape=[
            jax.ShapeDtypeStruct((B, P, 1), jnp.float32),
            jax.ShapeDtypeStruct((B, NBLK, 4, 128), jnp.float32),
        ],
        interpret=interpret,
    )(loc_data, loc_t, conf_data, conf_t3, mask_data, masks_t)

    out = pl.pallas_call(
        _select_body,
        out_shape=jax.ShapeDtypeStruct((8, 128), jnp.float32),
        interpret=interpret,
    )(s.reshape(B, P), sta)
    return (out[0, 0], out[1, 0], out[2, 0])


def kernel(loc_data, conf_data, mask_data, loc_t, conf_t, masks_t):
    return _run(loc_data, conf_data, mask_data, loc_t, conf_t, masks_t)
